# baseline (speedup 1.0000x reference)
import jax
import jax.numpy as jnp
from jax import lax
from jax.experimental import pallas as pl
from jax.experimental.pallas import tpu as pltpu

_C = 8


def _layer_stack(a_hat, d, h, layers):
    for W_ref, b_ref, act in layers:
        z = jnp.dot(h, W_ref[...], preferred_element_type=jnp.float32)
        zd = (z * d).astype(jnp.bfloat16)
        y = jnp.dot(a_hat, zd, preferred_element_type=jnp.float32)
        h = y * d + b_ref[...]
        if act:
            h = jnp.tanh(h)
    return h


def _gcn_body(x_ref, adj_ref, W0, b0, W1, b1, W2, b2, W3, b3, out_ref,
              buf, abf0, abf1, sem):
    B, N, _ = adj_ref.shape
    M = N // _C
    n_copies = B * _C

    def issue(g):
        if g < n_copies:
            b, c = divmod(g, _C)
            pltpu.make_async_copy(
                adj_ref.at[b, pl.ds(c * M, M), :],
                buf.at[g % 2],
                sem.at[g % 2],
            ).start()

    def consume(g, abf, rs_parts):
        b, c = divmod(g, _C)
        pltpu.make_async_copy(
            adj_ref.at[b, pl.ds(c * M, M), :],
            buf.at[g % 2],
            sem.at[g % 2],
        ).wait()
        chunk = buf[g % 2]
        rows = lax.broadcasted_iota(jnp.int32, (M, N), 0)
        cols = lax.broadcasted_iota(jnp.int32, (M, N), 1)
        fixed = jnp.where(cols == rows + c * M, 1.0, chunk)
        abf[c * M:(c + 1) * M, :] = fixed.astype(jnp.bfloat16)
        rs_parts.append(jnp.sum(fixed, axis=1, keepdims=True))
        issue(g + 2)

    layers = ((W0, b0, True), (W1, b1, True),
              (W2, b2, True), (W3, b3, False))

    issue(0)
    issue(1)

    rs0 = []
    for g in range(_C):
        consume(g, abf0, rs0)
    d0 = lax.rsqrt(jnp.maximum(jnp.concatenate(rs0, axis=0), 1.0))

    rs1 = []
    h0 = x_ref[0]
    for li, (W_ref, b_ref, act) in enumerate(layers):
        z = jnp.dot(h0, W_ref[...], preferred_element_type=jnp.float32)
        zd = (z * d0).astype(jnp.bfloat16)
        y = jnp.dot(abf0[...], zd, preferred_element_type=jnp.float32)
        h0 = y * d0 + b_ref[...]
        if act:
            h0 = jnp.tanh(h0)
        consume(_C + 2 * li, abf1, rs1)
        consume(_C + 2 * li + 1, abf1, rs1)
    out_ref[0] = h0

    d1 = lax.rsqrt(jnp.maximum(jnp.concatenate(rs1, axis=0), 1.0))
    out_ref[1] = _layer_stack(abf1[...], d1, x_ref[1], layers)


def kernel(x, adj, W0, b0, W1, b1, W2, b2, W3, b3):
    B, N, F_in = x.shape
    F_out = W3.shape[1]
    M = N // _C
    out = pl.pallas_call(
        _gcn_body,
        grid=(1,),
        in_specs=[
            pl.BlockSpec((B, N, F_in), lambda i: (0, 0, 0)),
            pl.BlockSpec(memory_space=pltpu.MemorySpace.HBM),
            pl.BlockSpec(W0.shape, lambda i: (0, 0)),
            pl.BlockSpec((1, W0.shape[1]), lambda i: (0, 0)),
            pl.BlockSpec(W1.shape, lambda i: (0, 0)),
            pl.BlockSpec((1, W1.shape[1]), lambda i: (0, 0)),
            pl.BlockSpec(W2.shape, lambda i: (0, 0)),
            pl.BlockSpec((1, W2.shape[1]), lambda i: (0, 0)),
            pl.BlockSpec(W3.shape, lambda i: (0, 0)),
            pl.BlockSpec((1, W3.shape[1]), lambda i: (0, 0)),
        ],
        out_specs=pl.BlockSpec((B, N, F_out), lambda i: (0, 0, 0)),
        out_shape=jax.ShapeDtypeStruct((B, N, F_out), jnp.float32),
        scratch_shapes=[
            pltpu.VMEM((2, M, N), jnp.float32),
            pltpu.VMEM((N, N), jnp.bfloat16),
            pltpu.VMEM((N, N), jnp.bfloat16),
            pltpu.SemaphoreType.DMA((2,)),
        ],
        compiler_params=pltpu.CompilerParams(
            dimension_semantics=("arbitrary",),
        ),
    )(x, adj, W0, b0.reshape(1, -1), W1, b1.reshape(1, -1),
      W2, b2.reshape(1, -1), W3, b3.reshape(1, -1))
    return out


# speedup vs baseline: 1.2152x; 1.2152x over previous
import jax
import jax.numpy as jnp
from jax import lax
from jax.experimental import pallas as pl
from jax.experimental.pallas import tpu as pltpu

_C = 8


def _gcn_body(x_ref, adj_ref, W0, b0, W1, b1, W2, b2, W3, b3, out_ref, abf):
    N = adj_ref.shape[1]
    M = N // _C

    rs_parts = []
    for c in range(_C):
        chunk = adj_ref[0, c * M:(c + 1) * M, :]
        rows = lax.broadcasted_iota(jnp.int32, (M, N), 0)
        cols = lax.broadcasted_iota(jnp.int32, (M, N), 1)
        fixed = jnp.where(cols == rows + c * M, 1.0, chunk)
        abf[c * M:(c + 1) * M, :] = fixed.astype(jnp.bfloat16)
        rs_parts.append(jnp.sum(fixed, axis=1, keepdims=True))
    rowsum = jnp.concatenate(rs_parts, axis=0)
    d = lax.rsqrt(jnp.maximum(rowsum, 1.0))

    a_hat = abf[...]
    h = x_ref[0]
    layers = ((W0, b0, True), (W1, b1, True),
              (W2, b2, True), (W3, b3, False))
    for W_ref, b_ref, act in layers:
        z = jnp.dot(h, W_ref[...], preferred_element_type=jnp.float32)
        zd = (z * d).astype(jnp.bfloat16)
        y = jnp.dot(a_hat, zd, preferred_element_type=jnp.float32)
        h = y * d + b_ref[...]
        if act:
            h = jnp.tanh(h)
    out_ref[0] = h


def kernel(x, adj, W0, b0, W1, b1, W2, b2, W3, b3):
    B, N, F_in = x.shape
    F_out = W3.shape[1]
    out = pl.pallas_call(
        _gcn_body,
        grid=(B,),
        in_specs=[
            pl.BlockSpec((1, N, F_in), lambda b: (b, 0, 0)),
            pl.BlockSpec((1, N, N), lambda b: (b, 0, 0)),
            pl.BlockSpec(W0.shape, lambda b: (0, 0)),
            pl.BlockSpec((1, W0.shape[1]), lambda b: (0, 0)),
            pl.BlockSpec(W1.shape, lambda b: (0, 0)),
            pl.BlockSpec((1, W1.shape[1]), lambda b: (0, 0)),
            pl.BlockSpec(W2.shape, lambda b: (0, 0)),
            pl.BlockSpec((1, W2.shape[1]), lambda b: (0, 0)),
            pl.BlockSpec(W3.shape, lambda b: (0, 0)),
            pl.BlockSpec((1, W3.shape[1]), lambda b: (0, 0)),
        ],
        out_specs=pl.BlockSpec((1, N, F_out), lambda b: (b, 0, 0)),
        out_shape=jax.ShapeDtypeStruct((B, N, F_out), jnp.float32),
        scratch_shapes=[pltpu.VMEM((N, N), jnp.bfloat16)],
        compiler_params=pltpu.CompilerParams(
            dimension_semantics=("arbitrary",),
            vmem_limit_bytes=57 * 1024 * 1024,
        ),
    )(x, adj, W0, b0.reshape(1, -1), W1, b1.reshape(1, -1),
      W2, b2.reshape(1, -1), W3, b3.reshape(1, -1))
    return out


# speedup vs baseline: 1.7815x; 1.4660x over previous
import jax
import jax.numpy as jnp
from jax import lax
from jax.experimental import pallas as pl
from jax.experimental.pallas import tpu as pltpu

_C = 8


def _gcn_body(x_ref, adj_ref, W0, b0, W1, b1, W2, b2, W3, b3, out_ref, abf):
    N = adj_ref.shape[1]
    M = N // _C

    rs_parts = []
    for c in range(_C):
        chunk = adj_ref[0, c * M:(c + 1) * M, :]
        rows = lax.broadcasted_iota(jnp.int32, (M, N), 0)
        cols = lax.broadcasted_iota(jnp.int32, (M, N), 1)
        fixed = jnp.where(cols == rows + c * M, 1.0, chunk)
        abf[c * M:(c + 1) * M, :] = fixed.astype(jnp.bfloat16)
        rs_parts.append(jnp.sum(fixed, axis=1, keepdims=True))
    rowsum = jnp.concatenate(rs_parts, axis=0)
    d = lax.rsqrt(jnp.maximum(rowsum, 1.0))

    h = x_ref[0]
    layers = ((W0, b0, True), (W1, b1, True),
              (W2, b2, True), (W3, b3, False))
    for W_ref, b_ref, act in layers:
        z = jnp.dot(h, W_ref[...], preferred_element_type=jnp.float32)
        zd = (z * d).astype(jnp.bfloat16)
        y_parts = [
            jnp.dot(abf[t * M:(t + 1) * M, :], zd,
                    preferred_element_type=jnp.float32)
            for t in range(_C)
        ]
        y = jnp.concatenate(y_parts, axis=0)
        h = y * d + b_ref[...]
        if act:
            h = jnp.tanh(h)
    out_ref[0] = h


def kernel(x, adj, W0, b0, W1, b1, W2, b2, W3, b3):
    B, N, F_in = x.shape
    F_out = W3.shape[1]
    out = pl.pallas_call(
        _gcn_body,
        grid=(B,),
        in_specs=[
            pl.BlockSpec((1, N, F_in), lambda b: (b, 0, 0)),
            pl.BlockSpec((1, N, N), lambda b: (b, 0, 0)),
            pl.BlockSpec(W0.shape, lambda b: (0, 0)),
            pl.BlockSpec((1, W0.shape[1]), lambda b: (0, 0)),
            pl.BlockSpec(W1.shape, lambda b: (0, 0)),
            pl.BlockSpec((1, W1.shape[1]), lambda b: (0, 0)),
            pl.BlockSpec(W2.shape, lambda b: (0, 0)),
            pl.BlockSpec((1, W2.shape[1]), lambda b: (0, 0)),
            pl.BlockSpec(W3.shape, lambda b: (0, 0)),
            pl.BlockSpec((1, W3.shape[1]), lambda b: (0, 0)),
        ],
        out_specs=pl.BlockSpec((1, N, F_out), lambda b: (b, 0, 0)),
        out_shape=jax.ShapeDtypeStruct((B, N, F_out), jnp.float32),
        scratch_shapes=[pltpu.VMEM((N, N), jnp.bfloat16)],
        compiler_params=pltpu.CompilerParams(
            dimension_semantics=("arbitrary",),
            vmem_limit_bytes=57 * 1024 * 1024,
        ),
    )(x, adj, W0, b0.reshape(1, -1), W1, b1.reshape(1, -1),
      W2, b2.reshape(1, -1), W3, b3.reshape(1, -1))
    return out


# speedup vs baseline: 1.7968x; 1.0086x over previous
import jax
import jax.numpy as jnp
from jax import lax
from jax.experimental import pallas as pl
from jax.experimental.pallas import tpu as pltpu

_C = 8
_MT = 16


def _gcn_body(x_ref, adj_ref, W0, b0, W1, b1, W2, b2, W3, b3, out_ref, abf):
    N = adj_ref.shape[1]
    M = N // _C

    rs_parts = []
    for c in range(_C):
        chunk = adj_ref[0, c * M:(c + 1) * M, :]
        rows = lax.broadcasted_iota(jnp.int32, (M, N), 0)
        cols = lax.broadcasted_iota(jnp.int32, (M, N), 1)
        fixed = jnp.where(cols == rows + c * M, 1.0, chunk)
        abf[c * M:(c + 1) * M, :] = fixed.astype(jnp.bfloat16)
        rs_parts.append(jnp.sum(fixed, axis=1, keepdims=True))
    rowsum = jnp.concatenate(rs_parts, axis=0)
    d = lax.rsqrt(jnp.maximum(rowsum, 1.0))

    h = x_ref[0]
    layers = ((W0, b0, True), (W1, b1, True),
              (W2, b2, True), (W3, b3, False))
    for W_ref, b_ref, act in layers:
        z = jnp.dot(h, W_ref[...], preferred_element_type=jnp.float32)
        zd = (z * d).astype(jnp.bfloat16)
        T = N // _MT
        y_parts = [
            jnp.dot(abf[t * T:(t + 1) * T, :], zd,
                    preferred_element_type=jnp.float32)
            for t in range(_MT)
        ]
        y = jnp.concatenate(y_parts, axis=0)
        h = y * d + b_ref[...]
        if act:
            h = jnp.tanh(h)
    out_ref[0] = h


def kernel(x, adj, W0, b0, W1, b1, W2, b2, W3, b3):
    B, N, F_in = x.shape
    F_out = W3.shape[1]
    out = pl.pallas_call(
        _gcn_body,
        grid=(B,),
        in_specs=[
            pl.BlockSpec((1, N, F_in), lambda b: (b, 0, 0)),
            pl.BlockSpec((1, N, N), lambda b: (b, 0, 0)),
            pl.BlockSpec(W0.shape, lambda b: (0, 0)),
            pl.BlockSpec((1, W0.shape[1]), lambda b: (0, 0)),
            pl.BlockSpec(W1.shape, lambda b: (0, 0)),
            pl.BlockSpec((1, W1.shape[1]), lambda b: (0, 0)),
            pl.BlockSpec(W2.shape, lambda b: (0, 0)),
            pl.BlockSpec((1, W2.shape[1]), lambda b: (0, 0)),
            pl.BlockSpec(W3.shape, lambda b: (0, 0)),
            pl.BlockSpec((1, W3.shape[1]), lambda b: (0, 0)),
        ],
        out_specs=pl.BlockSpec((1, N, F_out), lambda b: (b, 0, 0)),
        out_shape=jax.ShapeDtypeStruct((B, N, F_out), jnp.float32),
        scratch_shapes=[pltpu.VMEM((N, N), jnp.bfloat16)],
        compiler_params=pltpu.CompilerParams(
            dimension_semantics=("arbitrary",),
            vmem_limit_bytes=57 * 1024 * 1024,
        ),
    )(x, adj, W0, b0.reshape(1, -1), W1, b1.reshape(1, -1),
      W2, b2.reshape(1, -1), W3, b3.reshape(1, -1))
    return out


# speedup vs baseline: 1.7992x; 1.0014x over previous
import jax
import jax.numpy as jnp
from jax import lax
from jax.experimental import pallas as pl
from jax.experimental.pallas import tpu as pltpu

_C = 8
_MT = 16


def _gcn_body(x_ref, adj_ref, W0, b0, W1, b1, W2, b2, W3, b3, out_ref, abf):
    N = adj_ref.shape[1]
    M = N // _C

    rs_parts = []
    for c in range(_C):
        chunk = adj_ref[0, c * M:(c + 1) * M, :]
        rows = lax.broadcasted_iota(jnp.int32, (M, N), 0)
        cols = lax.broadcasted_iota(jnp.int32, (M, N), 1)
        fixed = jnp.where(cols == rows + c * M, 1.0, chunk)
        abf[c * M:(c + 1) * M, :] = fixed.astype(jnp.bfloat16)
        rs_parts.append(jnp.sum(fixed, axis=1, keepdims=True))
    rowsum = jnp.concatenate(rs_parts, axis=0)
    d = lax.rsqrt(jnp.maximum(rowsum, 1.0))

    h = x_ref[0]
    layers = ((W0, b0, True), (W1, b1, True),
              (W2, b2, True), (W3, b3, False))
    for W_ref, b_ref, act in layers:
        z = jnp.dot(h, W_ref[...], preferred_element_type=jnp.float32)
        zd = (z * d).astype(jnp.bfloat16)
        T = N // _MT
        h_parts = []
        for t in range(_MT):
            y_t = jnp.dot(abf[t * T:(t + 1) * T, :], zd,
                          preferred_element_type=jnp.float32)
            h_t = y_t * d[t * T:(t + 1) * T] + b_ref[...]
            h_parts.append(jnp.tanh(h_t) if act else h_t)
        h = jnp.concatenate(h_parts, axis=0)
    out_ref[0] = h


def kernel(x, adj, W0, b0, W1, b1, W2, b2, W3, b3):
    B, N, F_in = x.shape
    F_out = W3.shape[1]
    out = pl.pallas_call(
        _gcn_body,
        grid=(B,),
        in_specs=[
            pl.BlockSpec((1, N, F_in), lambda b: (b, 0, 0)),
            pl.BlockSpec((1, N, N), lambda b: (b, 0, 0)),
            pl.BlockSpec(W0.shape, lambda b: (0, 0)),
            pl.BlockSpec((1, W0.shape[1]), lambda b: (0, 0)),
            pl.BlockSpec(W1.shape, lambda b: (0, 0)),
            pl.BlockSpec((1, W1.shape[1]), lambda b: (0, 0)),
            pl.BlockSpec(W2.shape, lambda b: (0, 0)),
            pl.BlockSpec((1, W2.shape[1]), lambda b: (0, 0)),
            pl.BlockSpec(W3.shape, lambda b: (0, 0)),
            pl.BlockSpec((1, W3.shape[1]), lambda b: (0, 0)),
        ],
        out_specs=pl.BlockSpec((1, N, F_out), lambda b: (b, 0, 0)),
        out_shape=jax.ShapeDtypeStruct((B, N, F_out), jnp.float32),
        scratch_shapes=[pltpu.VMEM((N, N), jnp.bfloat16)],
        compiler_params=pltpu.CompilerParams(
            dimension_semantics=("arbitrary",),
            vmem_limit_bytes=57 * 1024 * 1024,
        ),
    )(x, adj, W0, b0.reshape(1, -1), W1, b1.reshape(1, -1),
      W2, b2.reshape(1, -1), W3, b3.reshape(1, -1))
    return out
